# baseline (speedup 1.0000x reference)
import functools

import jax
import jax.numpy as jnp
from jax import lax
from jax.experimental import pallas as pl
from jax.experimental.pallas import tpu as pltpu
from jax.experimental.pallas import tpu_sc as plsc


def _edge_kernel(he_ref, wt_ref, b_ref, nodes_ref, t0_ref, t1_ref, t2_ref, t3_ref, cc_ref):
    ht = jnp.dot(he_ref[...], wt_ref[...], preferred_element_type=jnp.float32) + b_ref[...]
    t0_ref[...] = ht[:, 0:128]
    t1_ref[...] = ht[:, 128:256]
    t2_ref[...] = ht[:, 256:384]
    t3_ref[...] = ht[:, 384:512]
    n = nodes_ref[...]
    rows, K = n.shape
    dup = jnp.zeros(n.shape, jnp.bool_)
    for d in range(1, K):
        shifted = jnp.concatenate(
            [jnp.full((rows, d), -1, jnp.int32), n[:, : K - d]], axis=1)
        dup = jnp.logical_or(dup, n == shifted)
    cc_ref[...] = 1.0 - dup.astype(jnp.float32)


def _node_kernel(x_ref, wt_ref, b_ref, ce_ref, s_ref):
    t = jnp.dot(x_ref[...], wt_ref[...], preferred_element_type=jnp.float32) + b_ref[...]
    lr = jnp.where(t >= 0, t, 0.01 * t)
    s_ref[...] = jnp.dot(lr, ce_ref[...], preferred_element_type=jnp.float32)


def _combine_kernel(H, S_ref, s_ref, cnt_ref, o_ref):
    s = s_ref[...]
    c = cnt_ref[...]
    m = jnp.maximum(s, 0.0)
    e = jnp.maximum(jnp.exp(s - m), 1e-35)
    denom = c * e + (float(H) - c) * jnp.exp(-m)
    a = e / denom
    o_ref[...] = jnp.maximum(a * S_ref[0], 0.0)


def _make_sc_scatter(SPAD, NPAD, n_tiles, blocks_per_tile, bw, stripe):
    s_stripe = SPAD // n_tiles
    mesh = plsc.VectorSubcoreMesh(core_axis_name="c", subcore_axis_name="s")

    @functools.partial(
        pl.kernel,
        mesh=mesh,
        out_type=[
            jax.ShapeDtypeStruct((4, SPAD, 128), jnp.float32),
            jax.ShapeDtypeStruct((NPAD,), jnp.float32),
        ],
        scratch_types=[
            pltpu.VMEM((blocks_per_tile, bw), jnp.int32),
            pltpu.VMEM((blocks_per_tile, bw), jnp.int32),
            pltpu.VMEM((blocks_per_tile, bw), jnp.float32),
            pltpu.VMEM((bw, 128), jnp.float32),
            pltpu.VMEM((bw, 128), jnp.float32),
            pltpu.VMEM((bw, 128), jnp.float32),
            pltpu.VMEM((bw, 128), jnp.float32),
            pltpu.VMEM_SHARED((SPAD, 128), jnp.float32),
            pltpu.VMEM_SHARED((NPAD,), jnp.float32),
            pltpu.SemaphoreType.DMA,
            pltpu.SemaphoreType.DMA,
            pltpu.SemaphoreType.DMA,
            pltpu.SemaphoreType.DMA,
            pltpu.SemaphoreType.DMA,
            pltpu.SemaphoreType.DMA,
            pltpu.SemaphoreType.DMA,
            pltpu.SemaphoreType.DMA,
        ],
    )
    def sc_scatter(nodes3, eids3, cntv3, zrows, z1, t0, t1, t2, t3,
                   S_out, cnt_out,
                   idxn_v, idxe_v, cval_v, rows_a, rows_b, rows_c, rows_d,
                   S_sh, cnt_sh,
                   ga, gb, gc, gd, sa, sb, sc_, sd):
        cid = lax.axis_index("c")
        sid = lax.axis_index("s")

        pltpu.sync_copy(nodes3.at[sid], idxn_v)
        pltpu.sync_copy(eids3.at[sid], idxe_v)

        @pl.when(cid == 0)
        def _():
            pltpu.sync_copy(z1, cnt_sh.at[pl.ds(sid * stripe, stripe)])
            plsc.subcore_barrier()
            pltpu.sync_copy(cntv3.at[sid], cval_v)

            def cbody(j, carry):
                pltpu.sync_copy(cval_v.at[j], cnt_sh.at[idxn_v.at[j]], add=True)
                return carry

            lax.fori_loop(0, blocks_per_tile, cbody, 0)
            plsc.subcore_barrier()
            pltpu.sync_copy(cnt_sh.at[pl.ds(sid * stripe, stripe)],
                            cnt_out.at[pl.ds(sid * stripe, stripe)])

        bufs = ((rows_a, ga, sa), (rows_b, gb, sb), (rows_c, gc, sc_))
        depth = len(bufs)
        n_iter = blocks_per_tile // depth
        for ti, table in enumerate((t0, t1, t2, t3)):
            @pl.when(cid == ti // 2)
            def _(table=table, ti=ti):
                pltpu.sync_copy(zrows, S_sh.at[pl.ds(sid * s_stripe, s_stripe)])
                plsc.subcore_barrier()

                for p, (rows, g, s) in enumerate(bufs):
                    pltpu.async_copy(table.at[idxe_v.at[p]], rows, g)

                def body(j, carry):
                    for p, (rows, g, s) in enumerate(bufs):
                        b = depth * j + p
                        pltpu.make_async_copy(table.at[idxe_v.at[b]], rows, g).wait()
                        pltpu.async_copy(rows, S_sh.at[idxn_v.at[b]], s, add=True)

                        @pl.when(j < n_iter - 1)
                        def _(rows=rows, g=g, s=s, b=b):
                            pltpu.make_async_copy(rows, S_sh.at[idxn_v.at[b]], s).wait()
                            pltpu.async_copy(table.at[idxe_v.at[b + depth]], rows, g)

                    return carry

                lax.fori_loop(0, n_iter, body, 0)
                for p, (rows, g, s) in enumerate(bufs):
                    b = blocks_per_tile - depth + p
                    pltpu.make_async_copy(rows, S_sh.at[idxn_v.at[b]], s).wait()
                plsc.subcore_barrier()
                pltpu.sync_copy(S_sh.at[pl.ds(sid * s_stripe, s_stripe)],
                                S_out.at[ti, pl.ds(sid * s_stripe, s_stripe)])

    return sc_scatter


def kernel(node_embeddings, hyperedge_embeddings, hyperedge_to_nodes, W_fc, b_fc, W_we, b_we, c_e):
    N, d_in = node_embeddings.shape
    H, K = hyperedge_to_nodes.shape
    d_out = W_fc.shape[0]
    assert d_in == 512 and d_out == 512

    n_tiles = 16
    stripe = 640
    NPAD = n_tiles * stripe
    PAIRS = H * K
    pairs_per_tile = PAIRS // n_tiles
    SPAD = 10112
    bw = 64
    blocks_per_tile = pairs_per_tile // bw

    f32 = jnp.float32

    eb = 256
    ht0, ht1, ht2, ht3, cc = pl.pallas_call(
        _edge_kernel,
        grid=(H // eb,),
        in_specs=[
            pl.BlockSpec((eb, d_in), lambda i: (i, 0)),
            pl.BlockSpec((d_in, d_out), lambda i: (0, 0)),
            pl.BlockSpec((1, d_out), lambda i: (0, 0)),
            pl.BlockSpec((eb, K), lambda i: (i, 0)),
        ],
        out_specs=[pl.BlockSpec((eb, 128), lambda i: (i, 0))] * 4
        + [pl.BlockSpec((eb, K), lambda i: (i, 0))],
        out_shape=[jax.ShapeDtypeStruct((H, 128), f32)] * 4
        + [jax.ShapeDtypeStruct((H, K), f32)],
    )(hyperedge_embeddings, W_fc.T, b_fc.reshape(1, -1), hyperedge_to_nodes)

    nb = 400
    s_col = pl.pallas_call(
        _node_kernel,
        grid=(N // nb,),
        in_specs=[
            pl.BlockSpec((nb, d_in), lambda i: (i, 0)),
            pl.BlockSpec((d_in, d_out), lambda i: (0, 0)),
            pl.BlockSpec((1, d_out), lambda i: (0, 0)),
            pl.BlockSpec((d_out, 1), lambda i: (0, 0)),
        ],
        out_specs=pl.BlockSpec((nb, 1), lambda i: (i, 0)),
        out_shape=jax.ShapeDtypeStruct((N, 1), f32),
    )(node_embeddings, W_we.T, b_we.reshape(1, -1), c_e.reshape(-1, 1))

    nodes3 = hyperedge_to_nodes.reshape(n_tiles, blocks_per_tile, bw)
    eids3 = (jnp.arange(PAIRS, dtype=jnp.int32) // K).reshape(
        n_tiles, blocks_per_tile, bw)
    cntv3 = cc.reshape(n_tiles, blocks_per_tile, bw)
    zrows = jnp.zeros((SPAD // n_tiles, 128), f32)
    z1 = jnp.zeros((stripe,), f32)

    sc = _make_sc_scatter(SPAD, NPAD, n_tiles, blocks_per_tile, bw, stripe)
    S_chunks, cnt = sc(nodes3, eids3, cntv3, zrows, z1, ht0, ht1, ht2, ht3)

    out = pl.pallas_call(
        functools.partial(_combine_kernel, H),
        grid=(N // nb, 4),
        in_specs=[
            pl.BlockSpec((1, nb, 128), lambda r, c: (c, r, 0)),
            pl.BlockSpec((nb, 1), lambda r, c: (r, 0)),
            pl.BlockSpec((nb, 1), lambda r, c: (r, 0)),
        ],
        out_specs=pl.BlockSpec((nb, 128), lambda r, c: (r, c)),
        out_shape=jax.ShapeDtypeStruct((N, d_out), f32),
    )(S_chunks, s_col, cnt[:N].reshape(N, 1))

    return out


# speedup vs baseline: 1.0974x; 1.0974x over previous
import functools

import jax
import jax.numpy as jnp
from jax import lax
from jax.experimental import pallas as pl
from jax.experimental.pallas import tpu as pltpu
from jax.experimental.pallas import tpu_sc as plsc


def _edge_kernel(he_ref, wt_ref, b_ref, nodes_ref, t0_ref, t1_ref, t2_ref, t3_ref, cc_ref):
    ht = jnp.dot(he_ref[...], wt_ref[...], preferred_element_type=jnp.float32) + b_ref[...]
    t0_ref[...] = ht[:, 0:128]
    t1_ref[...] = ht[:, 128:256]
    t2_ref[...] = ht[:, 256:384]
    t3_ref[...] = ht[:, 384:512]
    n = nodes_ref[...]
    rows, K = n.shape
    dup = jnp.zeros(n.shape, jnp.bool_)
    for d in range(1, K):
        shifted = jnp.concatenate(
            [jnp.full((rows, d), -1, jnp.int32), n[:, : K - d]], axis=1)
        dup = jnp.logical_or(dup, n == shifted)
    cc_ref[...] = 1.0 - dup.astype(jnp.float32)


def _node_kernel(x_ref, wt_ref, b_ref, ce_ref, s_ref):
    t = jnp.dot(x_ref[...], wt_ref[...], preferred_element_type=jnp.float32) + b_ref[...]
    lr = jnp.where(t >= 0, t, 0.01 * t)
    s_ref[...] = jnp.dot(lr, ce_ref[...], preferred_element_type=jnp.float32)


def _combine_kernel(H, S_ref, s_ref, cnt_ref, o_ref):
    s = s_ref[...]
    c = cnt_ref[...]
    m = jnp.maximum(s, 0.0)
    e = jnp.maximum(jnp.exp(s - m), 1e-35)
    denom = c * e + (float(H) - c) * jnp.exp(-m)
    a = e / denom
    o_ref[...] = jnp.maximum(a * S_ref[0], 0.0)


def _make_sc_scatter(SPAD, NPAD, n_tiles, blocks_per_tile, bw, stripe, pairs_per_tile):
    s_stripe = SPAD // n_tiles
    mesh = plsc.VectorSubcoreMesh(core_axis_name="c", subcore_axis_name="s")

    @functools.partial(
        pl.kernel,
        mesh=mesh,
        out_type=[
            jax.ShapeDtypeStruct((4, SPAD, 128), jnp.float32),
            jax.ShapeDtypeStruct((NPAD,), jnp.float32),
        ],
        scratch_types=[
            pltpu.VMEM((blocks_per_tile, bw), jnp.int32),
            pltpu.VMEM((blocks_per_tile, bw), jnp.int32),
            pltpu.VMEM((bw, 128), jnp.float32),
            pltpu.VMEM((bw, 128), jnp.float32),
            pltpu.VMEM((bw, 128), jnp.float32),
            pltpu.VMEM((bw, 128), jnp.float32),
            pltpu.VMEM_SHARED((SPAD, 128), jnp.float32),
            pltpu.VMEM_SHARED((NPAD,), jnp.float32),
            pltpu.SemaphoreType.DMA,
            pltpu.SemaphoreType.DMA,
            pltpu.SemaphoreType.DMA,
            pltpu.SemaphoreType.DMA,
            pltpu.SemaphoreType.DMA,
            pltpu.SemaphoreType.DMA,
            pltpu.SemaphoreType.DMA,
            pltpu.SemaphoreType.DMA,
        ],
    )
    def sc_scatter(nodes3, cntv3, zrows, z1, t0, t1, t2, t3,
                   S_out, cnt_out,
                   idxn_v, idxe_v, rows_a, rows_b, rows_c, rows_d,
                   S_sh, cnt_sh,
                   ga, gb, gc, gd, sa, sb, sc_, sd):
        cid = lax.axis_index("c")
        sid = lax.axis_index("s")

        pltpu.sync_copy(nodes3.at[sid], idxn_v)
        lanes = lax.iota(jnp.int32, 16)
        base0 = sid * pairs_per_tile

        def ibody(j, carry):
            for q in range(bw // 16):
                val = lax.shift_right_logical(base0 + j * bw + q * 16 + lanes, 5)
                idxe_v[j, pl.ds(q * 16, 16)] = val
            return carry

        lax.fori_loop(0, blocks_per_tile, ibody, 0)

        @pl.when(cid == 0)
        def _():
            pltpu.sync_copy(z1, cnt_sh.at[pl.ds(sid * stripe, stripe)])
            plsc.subcore_barrier()
            pltpu.sync_copy(cntv3.at[sid], rows_a.at[pl.ds(0, pairs_per_tile // 128)])

            def cbody(jj, carry):
                pltpu.sync_copy(rows_a.at[jj, pl.ds(0, bw)],
                                cnt_sh.at[idxn_v.at[2 * jj]], add=True)
                pltpu.sync_copy(rows_a.at[jj, pl.ds(bw, bw)],
                                cnt_sh.at[idxn_v.at[2 * jj + 1]], add=True)
                return carry

            lax.fori_loop(0, pairs_per_tile // 128, cbody, 0)
            plsc.subcore_barrier()
            pltpu.sync_copy(cnt_sh.at[pl.ds(sid * stripe, stripe)],
                            cnt_out.at[pl.ds(sid * stripe, stripe)])

        bufs = ((rows_a, ga, sa), (rows_b, gb, sb),
                (rows_c, gc, sc_), (rows_d, gd, sd))
        depth = len(bufs)
        assert blocks_per_tile % depth == 0
        n_iter = blocks_per_tile // depth
        for ti, table in enumerate((t0, t1, t2, t3)):
            @pl.when(cid == ti // 2)
            def _(table=table, ti=ti):
                pltpu.sync_copy(zrows, S_sh.at[pl.ds(sid * s_stripe, s_stripe)])
                plsc.subcore_barrier()

                for p, (rows, g, s) in enumerate(bufs):
                    pltpu.async_copy(table.at[idxe_v.at[p]], rows, g)

                def body(j, carry):
                    for p, (rows, g, s) in enumerate(bufs):
                        b = depth * j + p
                        pltpu.make_async_copy(table.at[idxe_v.at[b]], rows, g).wait()
                        pltpu.async_copy(rows, S_sh.at[idxn_v.at[b]], s, add=True)

                        @pl.when(j < n_iter - 1)
                        def _(rows=rows, g=g, s=s, b=b):
                            pltpu.make_async_copy(rows, S_sh.at[idxn_v.at[b]], s).wait()
                            pltpu.async_copy(table.at[idxe_v.at[b + depth]], rows, g)

                    return carry

                lax.fori_loop(0, n_iter, body, 0)
                for p, (rows, g, s) in enumerate(bufs):
                    b = blocks_per_tile - depth + p
                    pltpu.make_async_copy(rows, S_sh.at[idxn_v.at[b]], s).wait()
                plsc.subcore_barrier()
                pltpu.sync_copy(S_sh.at[pl.ds(sid * s_stripe, s_stripe)],
                                S_out.at[ti, pl.ds(sid * s_stripe, s_stripe)])

    return sc_scatter


def kernel(node_embeddings, hyperedge_embeddings, hyperedge_to_nodes, W_fc, b_fc, W_we, b_we, c_e):
    N, d_in = node_embeddings.shape
    H, K = hyperedge_to_nodes.shape
    d_out = W_fc.shape[0]
    assert d_in == 512 and d_out == 512

    n_tiles = 16
    stripe = 640
    NPAD = n_tiles * stripe
    PAIRS = H * K
    pairs_per_tile = PAIRS // n_tiles
    SPAD = 10112
    bw = 64
    blocks_per_tile = pairs_per_tile // bw

    f32 = jnp.float32

    eb = 256
    ht0, ht1, ht2, ht3, cc = pl.pallas_call(
        _edge_kernel,
        grid=(H // eb,),
        in_specs=[
            pl.BlockSpec((eb, d_in), lambda i: (i, 0)),
            pl.BlockSpec((d_in, d_out), lambda i: (0, 0)),
            pl.BlockSpec((1, d_out), lambda i: (0, 0)),
            pl.BlockSpec((eb, K), lambda i: (i, 0)),
        ],
        out_specs=[pl.BlockSpec((eb, 128), lambda i: (i, 0))] * 4
        + [pl.BlockSpec((eb, K), lambda i: (i, 0))],
        out_shape=[jax.ShapeDtypeStruct((H, 128), f32)] * 4
        + [jax.ShapeDtypeStruct((H, K), f32)],
    )(hyperedge_embeddings, W_fc.T, b_fc.reshape(1, -1), hyperedge_to_nodes)

    nb = 400
    s_col = pl.pallas_call(
        _node_kernel,
        grid=(N // nb,),
        in_specs=[
            pl.BlockSpec((nb, d_in), lambda i: (i, 0)),
            pl.BlockSpec((d_in, d_out), lambda i: (0, 0)),
            pl.BlockSpec((1, d_out), lambda i: (0, 0)),
            pl.BlockSpec((d_out, 1), lambda i: (0, 0)),
        ],
        out_specs=pl.BlockSpec((nb, 1), lambda i: (i, 0)),
        out_shape=jax.ShapeDtypeStruct((N, 1), f32),
    )(node_embeddings, W_we.T, b_we.reshape(1, -1), c_e.reshape(-1, 1))

    nodes3 = hyperedge_to_nodes.reshape(n_tiles, blocks_per_tile, bw)
    cntv3 = cc.reshape(n_tiles, pairs_per_tile // 128, 128)
    zrows = jnp.zeros((SPAD // n_tiles, 128), f32)
    z1 = jnp.zeros((stripe,), f32)

    sc = _make_sc_scatter(SPAD, NPAD, n_tiles, blocks_per_tile, bw, stripe,
                          pairs_per_tile)
    S_chunks, cnt = sc(nodes3, cntv3, zrows, z1, ht0, ht1, ht2, ht3)

    out = pl.pallas_call(
        functools.partial(_combine_kernel, H),
        grid=(N // nb, 4),
        in_specs=[
            pl.BlockSpec((1, nb, 128), lambda r, c: (c, r, 0)),
            pl.BlockSpec((nb, 1), lambda r, c: (r, 0)),
            pl.BlockSpec((nb, 1), lambda r, c: (r, 0)),
        ],
        out_specs=pl.BlockSpec((nb, 128), lambda r, c: (r, c)),
        out_shape=jax.ShapeDtypeStruct((N, d_out), f32),
    )(S_chunks, s_col, cnt[:N].reshape(N, 1))

    return out


# speedup vs baseline: 1.7384x; 1.5842x over previous
import functools

import jax
import jax.numpy as jnp
from jax import lax
from jax.experimental import pallas as pl
from jax.experimental.pallas import tpu as pltpu
from jax.experimental.pallas import tpu_sc as plsc


def _edge_kernel(he_ref, wt_ref, b_ref, nodes_ref, t0_ref, t1_ref, t2_ref, t3_ref, cc_ref):
    ht = jnp.dot(he_ref[...], wt_ref[...], preferred_element_type=jnp.float32) + b_ref[...]
    t0_ref[...] = ht[:, 0:128]
    t1_ref[...] = ht[:, 128:256]
    t2_ref[...] = ht[:, 256:384]
    t3_ref[...] = ht[:, 384:512]
    n = nodes_ref[...]
    rows, K = n.shape
    dup = jnp.zeros(n.shape, jnp.bool_)
    for d in range(1, K):
        shifted = jnp.concatenate(
            [jnp.full((rows, d), -1, jnp.int32), n[:, : K - d]], axis=1)
        dup = jnp.logical_or(dup, n == shifted)
    cc_ref[...] = 1.0 - dup.astype(jnp.float32)


def _node_kernel(x_ref, wt_ref, b_ref, ce_ref, s_ref):
    t = jnp.dot(x_ref[...], wt_ref[...], preferred_element_type=jnp.float32) + b_ref[...]
    lr = jnp.where(t >= 0, t, 0.01 * t)
    s_ref[...] = jnp.dot(lr, ce_ref[...], preferred_element_type=jnp.float32)


def _combine_kernel(H, S_ref, s_ref, cnt_ref, o_ref):
    s = s_ref[...]
    c = cnt_ref[...]
    m = jnp.maximum(s, 0.0)
    e = jnp.maximum(jnp.exp(s - m), 1e-35)
    denom = c * e + (float(H) - c) * jnp.exp(-m)
    a = e / denom
    o_ref[...] = jnp.maximum(a * S_ref[0], 0.0)


def _make_sc_scatter(SPAD, NPAD, n_tiles, blocks_per_tile, bw, stripe, pairs_per_tile):
    s_stripe = SPAD // n_tiles
    mesh = plsc.VectorSubcoreMesh(core_axis_name="c", subcore_axis_name="s")

    @functools.partial(
        pl.kernel,
        mesh=mesh,
        out_type=[
            jax.ShapeDtypeStruct((4, SPAD, 128), jnp.float32),
            jax.ShapeDtypeStruct((NPAD,), jnp.float32),
        ],
        scratch_types=[
            pltpu.VMEM((blocks_per_tile, bw), jnp.int32),
            pltpu.VMEM((128, 128), jnp.float32),
            pltpu.VMEM((bw, 128), jnp.float32),
            pltpu.VMEM((bw, 128), jnp.float32),
            pltpu.VMEM_SHARED((SPAD, 128), jnp.float32),
            pltpu.VMEM_SHARED((NPAD,), jnp.float32),
            pltpu.SemaphoreType.DMA,
            pltpu.SemaphoreType.DMA,
        ],
    )
    def sc_scatter(nodes3, cntv3, zrows, z1, t0, t1, t2, t3,
                   S_out, cnt_out,
                   idxn_v, rows_all, buf_a, buf_b,
                   S_sh, cnt_sh,
                   sa, sb):
        cid = lax.axis_index("c")
        sid = lax.axis_index("s")

        pltpu.sync_copy(nodes3.at[sid], idxn_v)

        @pl.when(cid == 0)
        def _():
            pltpu.sync_copy(z1, cnt_sh.at[pl.ds(sid * stripe, stripe)])
            plsc.subcore_barrier()
            pltpu.sync_copy(cntv3.at[sid], buf_a.at[pl.ds(0, pairs_per_tile // 128)])

            def cbody(jj, carry):
                pltpu.sync_copy(buf_a.at[jj, pl.ds(0, bw)],
                                cnt_sh.at[idxn_v.at[2 * jj]], add=True)
                pltpu.sync_copy(buf_a.at[jj, pl.ds(bw, bw)],
                                cnt_sh.at[idxn_v.at[2 * jj + 1]], add=True)
                return carry

            lax.fori_loop(0, pairs_per_tile // 128, cbody, 0)
            plsc.subcore_barrier()
            pltpu.sync_copy(cnt_sh.at[pl.ds(sid * stripe, stripe)],
                            cnt_out.at[pl.ds(sid * stripe, stripe)])

        epb = bw // 32

        def fill(buf, jj, half):
            for o in range(epb):
                src_row = 2 * epb * jj + epb * half + o
                vr = [rows_all[src_row, pl.ds(q * 16, 16)] for q in range(8)]
                for i in range(32):
                    for q in range(8):
                        buf[o * 32 + i, pl.ds(q * 16, 16)] = vr[q]

        for lp in range(2):
            @pl.when(cid == 0)
            def _(lp=lp):
                pltpu.sync_copy((t0, t1)[lp].at[pl.ds(sid * 128, 128)], rows_all)

            @pl.when(cid == 1)
            def _(lp=lp):
                pltpu.sync_copy((t2, t3)[lp].at[pl.ds(sid * 128, 128)], rows_all)

            pltpu.sync_copy(zrows, S_sh.at[pl.ds(sid * s_stripe, s_stripe)])
            plsc.subcore_barrier()

            fill(buf_a, 0, 0)
            pltpu.async_copy(buf_a, S_sh.at[idxn_v.at[0]], sa, add=True)
            fill(buf_b, 0, 1)
            pltpu.async_copy(buf_b, S_sh.at[idxn_v.at[1]], sb, add=True)

            def body(jj, carry):
                b0 = 2 * jj
                pltpu.make_async_copy(buf_a, S_sh.at[idxn_v.at[b0 - 2]], sa).wait()
                fill(buf_a, jj, 0)
                pltpu.async_copy(buf_a, S_sh.at[idxn_v.at[b0]], sa, add=True)
                pltpu.make_async_copy(buf_b, S_sh.at[idxn_v.at[b0 - 1]], sb).wait()
                fill(buf_b, jj, 1)
                pltpu.async_copy(buf_b, S_sh.at[idxn_v.at[b0 + 1]], sb, add=True)
                return carry

            lax.fori_loop(1, blocks_per_tile // 2, body, 0)
            pltpu.make_async_copy(
                buf_a, S_sh.at[idxn_v.at[blocks_per_tile - 2]], sa).wait()
            pltpu.make_async_copy(
                buf_b, S_sh.at[idxn_v.at[blocks_per_tile - 1]], sb).wait()
            plsc.subcore_barrier()

            @pl.when(cid == 0)
            def _(lp=lp):
                pltpu.sync_copy(S_sh.at[pl.ds(sid * s_stripe, s_stripe)],
                                S_out.at[lp, pl.ds(sid * s_stripe, s_stripe)])

            @pl.when(cid == 1)
            def _(lp=lp):
                pltpu.sync_copy(S_sh.at[pl.ds(sid * s_stripe, s_stripe)],
                                S_out.at[2 + lp, pl.ds(sid * s_stripe, s_stripe)])

    return sc_scatter


def kernel(node_embeddings, hyperedge_embeddings, hyperedge_to_nodes, W_fc, b_fc, W_we, b_we, c_e):
    N, d_in = node_embeddings.shape
    H, K = hyperedge_to_nodes.shape
    d_out = W_fc.shape[0]
    assert d_in == 512 and d_out == 512

    n_tiles = 16
    stripe = 640
    NPAD = n_tiles * stripe
    PAIRS = H * K
    pairs_per_tile = PAIRS // n_tiles
    SPAD = 10112
    bw = 64
    blocks_per_tile = pairs_per_tile // bw

    f32 = jnp.float32

    eb = 256
    ht0, ht1, ht2, ht3, cc = pl.pallas_call(
        _edge_kernel,
        grid=(H // eb,),
        in_specs=[
            pl.BlockSpec((eb, d_in), lambda i: (i, 0)),
            pl.BlockSpec((d_in, d_out), lambda i: (0, 0)),
            pl.BlockSpec((1, d_out), lambda i: (0, 0)),
            pl.BlockSpec((eb, K), lambda i: (i, 0)),
        ],
        out_specs=[pl.BlockSpec((eb, 128), lambda i: (i, 0))] * 4
        + [pl.BlockSpec((eb, K), lambda i: (i, 0))],
        out_shape=[jax.ShapeDtypeStruct((H, 128), f32)] * 4
        + [jax.ShapeDtypeStruct((H, K), f32)],
    )(hyperedge_embeddings, W_fc.T, b_fc.reshape(1, -1), hyperedge_to_nodes)

    nb = 400
    s_col = pl.pallas_call(
        _node_kernel,
        grid=(N // nb,),
        in_specs=[
            pl.BlockSpec((nb, d_in), lambda i: (i, 0)),
            pl.BlockSpec((d_in, d_out), lambda i: (0, 0)),
            pl.BlockSpec((1, d_out), lambda i: (0, 0)),
            pl.BlockSpec((d_out, 1), lambda i: (0, 0)),
        ],
        out_specs=pl.BlockSpec((nb, 1), lambda i: (i, 0)),
        out_shape=jax.ShapeDtypeStruct((N, 1), f32),
    )(node_embeddings, W_we.T, b_we.reshape(1, -1), c_e.reshape(-1, 1))

    nodes3 = hyperedge_to_nodes.reshape(n_tiles, blocks_per_tile, bw)
    cntv3 = cc.reshape(n_tiles, pairs_per_tile // 128, 128)
    zrows = jnp.zeros((SPAD // n_tiles, 128), f32)
    z1 = jnp.zeros((stripe,), f32)

    sc = _make_sc_scatter(SPAD, NPAD, n_tiles, blocks_per_tile, bw, stripe,
                          pairs_per_tile)
    S_chunks, cnt = sc(nodes3, cntv3, zrows, z1, ht0, ht1, ht2, ht3)

    out = pl.pallas_call(
        functools.partial(_combine_kernel, H),
        grid=(N // nb, 4),
        in_specs=[
            pl.BlockSpec((1, nb, 128), lambda r, c: (c, r, 0)),
            pl.BlockSpec((nb, 1), lambda r, c: (r, 0)),
            pl.BlockSpec((nb, 1), lambda r, c: (r, 0)),
        ],
        out_specs=pl.BlockSpec((nb, 128), lambda r, c: (r, c)),
        out_shape=jax.ShapeDtypeStruct((N, d_out), f32),
    )(S_chunks, s_col, cnt[:N].reshape(N, 1))

    return out


# speedup vs baseline: 2.1758x; 1.2516x over previous
import functools

import jax
import jax.numpy as jnp
from jax import lax
from jax.experimental import pallas as pl
from jax.experimental.pallas import tpu as pltpu
from jax.experimental.pallas import tpu_sc as plsc


def _edge_kernel(he_ref, wt_ref, b_ref, nodes_ref, t0_ref, t1_ref, t2_ref, t3_ref, cc_ref):
    ht = jnp.dot(he_ref[...], wt_ref[...], preferred_element_type=jnp.float32) + b_ref[...]
    t0_ref[...] = ht[:, 0:128]
    t1_ref[...] = ht[:, 128:256]
    t2_ref[...] = ht[:, 256:384]
    t3_ref[...] = ht[:, 384:512]
    n = nodes_ref[...]
    rows, K = n.shape
    dup = jnp.zeros(n.shape, jnp.bool_)
    for d in range(1, K):
        shifted = jnp.concatenate(
            [jnp.full((rows, d), -1, jnp.int32), n[:, : K - d]], axis=1)
        dup = jnp.logical_or(dup, n == shifted)
    cc_ref[...] = 1.0 - dup.astype(jnp.float32)


def _node_kernel(x_ref, wt_ref, b_ref, ce_ref, s_ref):
    t = jnp.dot(x_ref[...], wt_ref[...], preferred_element_type=jnp.float32) + b_ref[...]
    lr = jnp.where(t >= 0, t, 0.01 * t)
    s_ref[...] = jnp.dot(lr, ce_ref[...], preferred_element_type=jnp.float32)


def _combine_kernel(H, S_ref, s_ref, cnt_ref, o_ref):
    s = s_ref[...]
    c = cnt_ref[...]
    m = jnp.maximum(s, 0.0)
    e = jnp.maximum(jnp.exp(s - m), 1e-35)
    denom = c * e + (float(H) - c) * jnp.exp(-m)
    a = e / denom
    S4 = S_ref[...]
    S = jnp.concatenate([S4[0], S4[1], S4[2], S4[3]], axis=1)
    o_ref[...] = jnp.maximum(a * S, 0.0)


def _make_sc_scatter(SPAD, NPAD, n_tiles, blocks_per_tile, bw, stripe, pairs_per_tile):
    s_stripe = SPAD // n_tiles
    mesh = plsc.VectorSubcoreMesh(core_axis_name="c", subcore_axis_name="s")

    @functools.partial(
        pl.kernel,
        mesh=mesh,
        out_type=[
            jax.ShapeDtypeStruct((4, SPAD, 128), jnp.float32),
            jax.ShapeDtypeStruct((NPAD,), jnp.float32),
        ],
        scratch_types=[
            pltpu.VMEM((blocks_per_tile, bw), jnp.int32),
            pltpu.VMEM((128, 128), jnp.float32),
            pltpu.VMEM((bw, 128), jnp.float32),
            pltpu.VMEM((bw, 128), jnp.float32),
            pltpu.VMEM_SHARED((SPAD, 128), jnp.float32),
            pltpu.VMEM_SHARED((NPAD,), jnp.float32),
            pltpu.SemaphoreType.DMA,
            pltpu.SemaphoreType.DMA,
        ],
    )
    def sc_scatter(nodes3, cntv3, zrows, z1, t0, t1, t2, t3,
                   S_out, cnt_out,
                   idxn_v, rows_all, buf_a, buf_b,
                   S_sh, cnt_sh,
                   sa, sb):
        cid = lax.axis_index("c")
        sid = lax.axis_index("s")

        pltpu.sync_copy(nodes3.at[sid], idxn_v)

        @pl.when(cid == 0)
        def _():
            pltpu.sync_copy(z1, cnt_sh.at[pl.ds(sid * stripe, stripe)])
            plsc.subcore_barrier()
            pltpu.sync_copy(cntv3.at[sid], buf_a.at[pl.ds(0, pairs_per_tile // 128)])

            def cbody(jj, carry):
                pltpu.sync_copy(buf_a.at[jj, pl.ds(0, bw)],
                                cnt_sh.at[idxn_v.at[2 * jj]], add=True)
                pltpu.sync_copy(buf_a.at[jj, pl.ds(bw, bw)],
                                cnt_sh.at[idxn_v.at[2 * jj + 1]], add=True)
                return carry

            lax.fori_loop(0, pairs_per_tile // 128, cbody, 0)
            plsc.subcore_barrier()
            pltpu.sync_copy(cnt_sh.at[pl.ds(sid * stripe, stripe)],
                            cnt_out.at[pl.ds(sid * stripe, stripe)])

        epb = bw // 32

        def fill(buf, jj, half):
            for o in range(epb):
                src_row = 2 * epb * jj + epb * half + o
                vr = [rows_all[src_row, pl.ds(q * 16, 16)] for q in range(8)]
                for i in range(32):
                    for q in range(8):
                        buf[o * 32 + i, pl.ds(q * 16, 16)] = vr[q]

        for lp in range(2):
            @pl.when(cid == 0)
            def _(lp=lp):
                pltpu.sync_copy((t0, t1)[lp].at[pl.ds(sid * 128, 128)], rows_all)

            @pl.when(cid == 1)
            def _(lp=lp):
                pltpu.sync_copy((t2, t3)[lp].at[pl.ds(sid * 128, 128)], rows_all)

            pltpu.sync_copy(zrows, S_sh.at[pl.ds(sid * s_stripe, s_stripe)])
            plsc.subcore_barrier()

            fill(buf_a, 0, 0)
            pltpu.async_copy(buf_a, S_sh.at[idxn_v.at[0]], sa, add=True)
            fill(buf_b, 0, 1)
            pltpu.async_copy(buf_b, S_sh.at[idxn_v.at[1]], sb, add=True)

            def body(jj, carry):
                b0 = 2 * jj
                pltpu.make_async_copy(buf_a, S_sh.at[idxn_v.at[b0 - 2]], sa).wait()
                fill(buf_a, jj, 0)
                pltpu.async_copy(buf_a, S_sh.at[idxn_v.at[b0]], sa, add=True)
                pltpu.make_async_copy(buf_b, S_sh.at[idxn_v.at[b0 - 1]], sb).wait()
                fill(buf_b, jj, 1)
                pltpu.async_copy(buf_b, S_sh.at[idxn_v.at[b0 + 1]], sb, add=True)
                return carry

            lax.fori_loop(1, blocks_per_tile // 2, body, 0)
            pltpu.make_async_copy(
                buf_a, S_sh.at[idxn_v.at[blocks_per_tile - 2]], sa).wait()
            pltpu.make_async_copy(
                buf_b, S_sh.at[idxn_v.at[blocks_per_tile - 1]], sb).wait()
            plsc.subcore_barrier()

            @pl.when(cid == 0)
            def _(lp=lp):
                pltpu.sync_copy(S_sh.at[pl.ds(sid * s_stripe, s_stripe)],
                                S_out.at[lp, pl.ds(sid * s_stripe, s_stripe)])

            @pl.when(cid == 1)
            def _(lp=lp):
                pltpu.sync_copy(S_sh.at[pl.ds(sid * s_stripe, s_stripe)],
                                S_out.at[2 + lp, pl.ds(sid * s_stripe, s_stripe)])

    return sc_scatter


def kernel(node_embeddings, hyperedge_embeddings, hyperedge_to_nodes, W_fc, b_fc, W_we, b_we, c_e):
    N, d_in = node_embeddings.shape
    H, K = hyperedge_to_nodes.shape
    d_out = W_fc.shape[0]
    assert d_in == 512 and d_out == 512

    n_tiles = 16
    stripe = 640
    NPAD = n_tiles * stripe
    PAIRS = H * K
    pairs_per_tile = PAIRS // n_tiles
    SPAD = 10112
    bw = 64
    blocks_per_tile = pairs_per_tile // bw

    f32 = jnp.float32

    eb = 256
    ht0, ht1, ht2, ht3, cc = pl.pallas_call(
        _edge_kernel,
        grid=(H // eb,),
        in_specs=[
            pl.BlockSpec((eb, d_in), lambda i: (i, 0)),
            pl.BlockSpec((d_in, d_out), lambda i: (0, 0)),
            pl.BlockSpec((1, d_out), lambda i: (0, 0)),
            pl.BlockSpec((eb, K), lambda i: (i, 0)),
        ],
        out_specs=[pl.BlockSpec((eb, 128), lambda i: (i, 0))] * 4
        + [pl.BlockSpec((eb, K), lambda i: (i, 0))],
        out_shape=[jax.ShapeDtypeStruct((H, 128), f32)] * 4
        + [jax.ShapeDtypeStruct((H, K), f32)],
    )(hyperedge_embeddings, W_fc.T, b_fc.reshape(1, -1), hyperedge_to_nodes)

    nb = 400
    s_col = pl.pallas_call(
        _node_kernel,
        grid=(N // nb,),
        in_specs=[
            pl.BlockSpec((nb, d_in), lambda i: (i, 0)),
            pl.BlockSpec((d_in, d_out), lambda i: (0, 0)),
            pl.BlockSpec((1, d_out), lambda i: (0, 0)),
            pl.BlockSpec((d_out, 1), lambda i: (0, 0)),
        ],
        out_specs=pl.BlockSpec((nb, 1), lambda i: (i, 0)),
        out_shape=jax.ShapeDtypeStruct((N, 1), f32),
    )(node_embeddings, W_we.T, b_we.reshape(1, -1), c_e.reshape(-1, 1))

    nodes3 = hyperedge_to_nodes.reshape(n_tiles, blocks_per_tile, bw)
    cntv3 = cc.reshape(n_tiles, pairs_per_tile // 128, 128)
    zrows = jnp.zeros((SPAD // n_tiles, 128), f32)
    z1 = jnp.zeros((stripe,), f32)

    sc = _make_sc_scatter(SPAD, NPAD, n_tiles, blocks_per_tile, bw, stripe,
                          pairs_per_tile)
    S_chunks, cnt = sc(nodes3, cntv3, zrows, z1, ht0, ht1, ht2, ht3)

    out = pl.pallas_call(
        functools.partial(_combine_kernel, H),
        grid=(N // nb,),
        in_specs=[
            pl.BlockSpec((4, nb, 128), lambda r: (0, r, 0)),
            pl.BlockSpec((nb, 1), lambda r: (r, 0)),
            pl.BlockSpec((nb, 1), lambda r: (r, 0)),
        ],
        out_specs=pl.BlockSpec((nb, d_out), lambda r: (r, 0)),
        out_shape=jax.ShapeDtypeStruct((N, d_out), f32),
    )(S_chunks, s_col, cnt[:N].reshape(N, 1))

    return out


# speedup vs baseline: 2.2078x; 1.0147x over previous
import functools

import jax
import jax.numpy as jnp
from jax import lax
from jax.experimental import pallas as pl
from jax.experimental.pallas import tpu as pltpu
from jax.experimental.pallas import tpu_sc as plsc


def _edge_kernel(he_ref, wt_ref, b_ref, nodes_ref, t0_ref, t1_ref, t2_ref, t3_ref, cc_ref):
    ht = jnp.dot(he_ref[...], wt_ref[...], preferred_element_type=jnp.float32) + b_ref[...]
    t0_ref[...] = ht[:, 0:128]
    t1_ref[...] = ht[:, 128:256]
    t2_ref[...] = ht[:, 256:384]
    t3_ref[...] = ht[:, 384:512]
    n = nodes_ref[...]
    rows, K = n.shape
    eq = n[:, :, None] == n[:, None, :]
    kk = jax.lax.broadcasted_iota(jnp.int32, (rows, K, K), 1)
    kp = jax.lax.broadcasted_iota(jnp.int32, (rows, K, K), 2)
    dup = jnp.any(jnp.logical_and(eq, kp < kk), axis=2)
    cc_ref[...] = 1.0 - dup.astype(jnp.float32)


def _node_kernel(x_ref, wt_ref, b_ref, ce_ref, s_ref):
    t = jnp.dot(x_ref[...], wt_ref[...], preferred_element_type=jnp.float32) + b_ref[...]
    lr = jnp.where(t >= 0, t, 0.01 * t)
    s_ref[...] = jnp.dot(lr, ce_ref[...], preferred_element_type=jnp.float32)


def _combine_kernel(H, S_ref, s_ref, cnt_ref, o_ref):
    s = s_ref[...]
    c = cnt_ref[...]
    m = jnp.maximum(s, 0.0)
    e = jnp.maximum(jnp.exp(s - m), 1e-35)
    denom = c * e + (float(H) - c) * jnp.exp(-m)
    a = e / denom
    S4 = S_ref[...]
    S = jnp.concatenate([S4[0], S4[1], S4[2], S4[3]], axis=1)
    o_ref[...] = jnp.maximum(a * S, 0.0)


def _make_sc_scatter(SPAD, NPAD, n_tiles, blocks_per_tile, bw, stripe, pairs_per_tile):
    s_stripe = SPAD // n_tiles
    mesh = plsc.VectorSubcoreMesh(core_axis_name="c", subcore_axis_name="s")

    @functools.partial(
        pl.kernel,
        mesh=mesh,
        out_type=[
            jax.ShapeDtypeStruct((4, SPAD, 128), jnp.float32),
            jax.ShapeDtypeStruct((NPAD,), jnp.float32),
        ],
        scratch_types=[
            pltpu.VMEM((blocks_per_tile, bw), jnp.int32),
            pltpu.VMEM((128, 128), jnp.float32),
            pltpu.VMEM((bw, 128), jnp.float32),
            pltpu.VMEM((bw, 128), jnp.float32),
            pltpu.VMEM_SHARED((SPAD, 128), jnp.float32),
            pltpu.VMEM_SHARED((NPAD,), jnp.float32),
            pltpu.SemaphoreType.DMA,
            pltpu.SemaphoreType.DMA,
        ],
    )
    def sc_scatter(nodes3, cntv3, zrows, z1, t0, t1, t2, t3,
                   S_out, cnt_out,
                   idxn_v, rows_all, buf_a, buf_b,
                   S_sh, cnt_sh,
                   sa, sb):
        cid = lax.axis_index("c")
        sid = lax.axis_index("s")

        pltpu.sync_copy(nodes3.at[sid], idxn_v)

        @pl.when(cid == 0)
        def _():
            pltpu.sync_copy(z1, cnt_sh.at[pl.ds(sid * stripe, stripe)])
            plsc.subcore_barrier()
            pltpu.sync_copy(cntv3.at[sid], buf_a.at[pl.ds(0, pairs_per_tile // 128)])

            def cbody(jj, carry):
                pltpu.sync_copy(buf_a.at[jj, pl.ds(0, bw)],
                                cnt_sh.at[idxn_v.at[2 * jj]], add=True)
                pltpu.sync_copy(buf_a.at[jj, pl.ds(bw, bw)],
                                cnt_sh.at[idxn_v.at[2 * jj + 1]], add=True)
                return carry

            lax.fori_loop(0, pairs_per_tile // 128, cbody, 0)
            plsc.subcore_barrier()
            pltpu.sync_copy(cnt_sh.at[pl.ds(sid * stripe, stripe)],
                            cnt_out.at[pl.ds(sid * stripe, stripe)])

        epb = bw // 32

        def fill(buf, jj, half):
            for o in range(epb):
                src_row = 2 * epb * jj + epb * half + o
                vr = [rows_all[src_row, pl.ds(q * 16, 16)] for q in range(8)]
                for i in range(32):
                    for q in range(8):
                        buf[o * 32 + i, pl.ds(q * 16, 16)] = vr[q]

        for lp in range(2):
            @pl.when(cid == 0)
            def _(lp=lp):
                pltpu.sync_copy((t0, t1)[lp].at[pl.ds(sid * 128, 128)], rows_all)

            @pl.when(cid == 1)
            def _(lp=lp):
                pltpu.sync_copy((t2, t3)[lp].at[pl.ds(sid * 128, 128)], rows_all)

            pltpu.sync_copy(zrows, S_sh.at[pl.ds(sid * s_stripe, s_stripe)])
            plsc.subcore_barrier()

            fill(buf_a, 0, 0)
            pltpu.async_copy(buf_a, S_sh.at[idxn_v.at[0]], sa, add=True)
            fill(buf_b, 0, 1)
            pltpu.async_copy(buf_b, S_sh.at[idxn_v.at[1]], sb, add=True)

            def body(jj, carry):
                b0 = 2 * jj
                pltpu.make_async_copy(buf_a, S_sh.at[idxn_v.at[b0 - 2]], sa).wait()
                fill(buf_a, jj, 0)
                pltpu.async_copy(buf_a, S_sh.at[idxn_v.at[b0]], sa, add=True)
                pltpu.make_async_copy(buf_b, S_sh.at[idxn_v.at[b0 - 1]], sb).wait()
                fill(buf_b, jj, 1)
                pltpu.async_copy(buf_b, S_sh.at[idxn_v.at[b0 + 1]], sb, add=True)
                return carry

            lax.fori_loop(1, blocks_per_tile // 2, body, 0)
            pltpu.make_async_copy(
                buf_a, S_sh.at[idxn_v.at[blocks_per_tile - 2]], sa).wait()
            pltpu.make_async_copy(
                buf_b, S_sh.at[idxn_v.at[blocks_per_tile - 1]], sb).wait()
            plsc.subcore_barrier()

            @pl.when(cid == 0)
            def _(lp=lp):
                pltpu.sync_copy(S_sh.at[pl.ds(sid * s_stripe, s_stripe)],
                                S_out.at[lp, pl.ds(sid * s_stripe, s_stripe)])

            @pl.when(cid == 1)
            def _(lp=lp):
                pltpu.sync_copy(S_sh.at[pl.ds(sid * s_stripe, s_stripe)],
                                S_out.at[2 + lp, pl.ds(sid * s_stripe, s_stripe)])

    return sc_scatter


def kernel(node_embeddings, hyperedge_embeddings, hyperedge_to_nodes, W_fc, b_fc, W_we, b_we, c_e):
    N, d_in = node_embeddings.shape
    H, K = hyperedge_to_nodes.shape
    d_out = W_fc.shape[0]
    assert d_in == 512 and d_out == 512

    n_tiles = 16
    stripe = 640
    NPAD = n_tiles * stripe
    PAIRS = H * K
    pairs_per_tile = PAIRS // n_tiles
    SPAD = 10112
    bw = 64
    blocks_per_tile = pairs_per_tile // bw

    f32 = jnp.float32

    eb = 256
    ht0, ht1, ht2, ht3, cc = pl.pallas_call(
        _edge_kernel,
        grid=(H // eb,),
        in_specs=[
            pl.BlockSpec((eb, d_in), lambda i: (i, 0)),
            pl.BlockSpec((d_in, d_out), lambda i: (0, 0)),
            pl.BlockSpec((1, d_out), lambda i: (0, 0)),
            pl.BlockSpec((eb, K), lambda i: (i, 0)),
        ],
        out_specs=[pl.BlockSpec((eb, 128), lambda i: (i, 0))] * 4
        + [pl.BlockSpec((eb, K), lambda i: (i, 0))],
        out_shape=[jax.ShapeDtypeStruct((H, 128), f32)] * 4
        + [jax.ShapeDtypeStruct((H, K), f32)],
    )(hyperedge_embeddings, W_fc.T, b_fc.reshape(1, -1), hyperedge_to_nodes)

    nb = 400
    s_col = pl.pallas_call(
        _node_kernel,
        grid=(N // nb,),
        in_specs=[
            pl.BlockSpec((nb, d_in), lambda i: (i, 0)),
            pl.BlockSpec((d_in, d_out), lambda i: (0, 0)),
            pl.BlockSpec((1, d_out), lambda i: (0, 0)),
            pl.BlockSpec((d_out, 1), lambda i: (0, 0)),
        ],
        out_specs=pl.BlockSpec((nb, 1), lambda i: (i, 0)),
        out_shape=jax.ShapeDtypeStruct((N, 1), f32),
    )(node_embeddings, W_we.T, b_we.reshape(1, -1), c_e.reshape(-1, 1))

    nodes3 = hyperedge_to_nodes.reshape(n_tiles, blocks_per_tile, bw)
    cntv3 = cc.reshape(n_tiles, pairs_per_tile // 128, 128)
    zrows = jnp.zeros((SPAD // n_tiles, 128), f32)
    z1 = jnp.zeros((stripe,), f32)

    sc = _make_sc_scatter(SPAD, NPAD, n_tiles, blocks_per_tile, bw, stripe,
                          pairs_per_tile)
    S_chunks, cnt = sc(nodes3, cntv3, zrows, z1, ht0, ht1, ht2, ht3)

    cb = 1000
    out = pl.pallas_call(
        functools.partial(_combine_kernel, H),
        grid=(N // cb,),
        in_specs=[
            pl.BlockSpec((4, cb, 128), lambda r: (0, r, 0)),
            pl.BlockSpec((cb, 1), lambda r: (r, 0)),
            pl.BlockSpec((cb, 1), lambda r: (r, 0)),
        ],
        out_specs=pl.BlockSpec((cb, d_out), lambda r: (r, 0)),
        out_shape=jax.ShapeDtypeStruct((N, d_out), f32),
    )(S_chunks, s_col, cnt[:N].reshape(N, 1))

    return out


# speedup vs baseline: 2.2579x; 1.0227x over previous
import functools

import jax
import jax.numpy as jnp
from jax import lax
from jax.experimental import pallas as pl
from jax.experimental.pallas import tpu as pltpu
from jax.experimental.pallas import tpu_sc as plsc


def _edge_kernel(he_ref, wt_ref, b_ref, nodes_ref, t0_ref, t1_ref, t2_ref, t3_ref, cc_ref):
    ht = jnp.dot(he_ref[...], wt_ref[...], preferred_element_type=jnp.float32) + b_ref[...]
    t0_ref[...] = ht[:, 0:128]
    t1_ref[...] = ht[:, 128:256]
    t2_ref[...] = ht[:, 256:384]
    t3_ref[...] = ht[:, 384:512]
    n = nodes_ref[...]
    rows, K = n.shape
    eq = n[:, :, None] == n[:, None, :]
    kk = jax.lax.broadcasted_iota(jnp.int32, (rows, K, K), 1)
    kp = jax.lax.broadcasted_iota(jnp.int32, (rows, K, K), 2)
    dup = jnp.any(jnp.logical_and(eq, kp < kk), axis=2)
    cc_ref[...] = 1.0 - dup.astype(jnp.float32)


def _node_kernel(x_ref, wt_ref, b_ref, ce_ref, s_ref):
    t = jnp.dot(x_ref[...], wt_ref[...], preferred_element_type=jnp.float32) + b_ref[...]
    lr = jnp.where(t >= 0, t, 0.01 * t)
    s_ref[...] = jnp.dot(lr, ce_ref[...], preferred_element_type=jnp.float32)


def _combine_kernel(H, S_ref, s_ref, cnt_ref, o_ref):
    s = s_ref[...]
    c = cnt_ref[...]
    m = jnp.maximum(s, 0.0)
    e = jnp.maximum(jnp.exp(s - m), 1e-35)
    denom = c * e + (float(H) - c) * jnp.exp(-m)
    a = e / denom
    S4 = S_ref[...]
    S = jnp.concatenate([S4[0], S4[1], S4[2], S4[3]], axis=1)
    o_ref[...] = jnp.maximum(a * S, 0.0)


def _make_sc_scatter(SPAD, NPAD, n_tiles, blocks_per_tile, bw, stripe, pairs_per_tile):
    s_stripe = SPAD // n_tiles
    mesh = plsc.VectorSubcoreMesh(core_axis_name="c", subcore_axis_name="s")

    @functools.partial(
        pl.kernel,
        mesh=mesh,
        out_type=[
            jax.ShapeDtypeStruct((4, SPAD, 128), jnp.float32),
            jax.ShapeDtypeStruct((NPAD,), jnp.float32),
        ],
        scratch_types=[
            pltpu.VMEM((blocks_per_tile, bw), jnp.int32),
            pltpu.VMEM((128, 128), jnp.float32),
            pltpu.VMEM((bw, 128), jnp.float32),
            pltpu.VMEM((bw, 128), jnp.float32),
            pltpu.VMEM_SHARED((SPAD, 128), jnp.float32),
            pltpu.VMEM_SHARED((NPAD,), jnp.float32),
            pltpu.SemaphoreType.DMA,
            pltpu.SemaphoreType.DMA,
        ],
    )
    def sc_scatter(nodes3, cntv3, zrows, z1, t0, t1, t2, t3,
                   S_out, cnt_out,
                   idxn_v, rows_all, buf_a, buf_b,
                   S_sh, cnt_sh,
                   sa, sb):
        cid = lax.axis_index("c")
        sid = lax.axis_index("s")

        pltpu.sync_copy(nodes3.at[sid], idxn_v)

        @pl.when(cid == 0)
        def _():
            pltpu.sync_copy(z1, cnt_sh.at[pl.ds(sid * stripe, stripe)])
            plsc.subcore_barrier()
            pltpu.sync_copy(cntv3.at[sid], buf_a.at[pl.ds(0, pairs_per_tile // 128)])

            def cbody(jj, carry):
                pltpu.async_copy(buf_a.at[jj, pl.ds(0, bw)],
                                 cnt_sh.at[idxn_v.at[2 * jj]], sa, add=True)
                pltpu.async_copy(buf_a.at[jj, pl.ds(bw, bw)],
                                 cnt_sh.at[idxn_v.at[2 * jj + 1]], sb, add=True)
                return carry

            lax.fori_loop(0, pairs_per_tile // 128, cbody, 0)

            def cdrain(jj, carry):
                pltpu.make_async_copy(buf_a.at[jj, pl.ds(0, bw)],
                                      cnt_sh.at[idxn_v.at[2 * jj]], sa).wait()
                pltpu.make_async_copy(buf_a.at[jj, pl.ds(bw, bw)],
                                      cnt_sh.at[idxn_v.at[2 * jj + 1]], sb).wait()
                return carry

            lax.fori_loop(0, pairs_per_tile // 128, cdrain, 0)
            plsc.subcore_barrier()
            pltpu.sync_copy(cnt_sh.at[pl.ds(sid * stripe, stripe)],
                            cnt_out.at[pl.ds(sid * stripe, stripe)])

        epb = bw // 32

        def fill(buf, jj, half):
            for o in range(epb):
                src_row = 2 * epb * jj + epb * half + o
                vr = [rows_all[src_row, pl.ds(q * 16, 16)] for q in range(8)]
                for i in range(32):
                    for q in range(8):
                        buf[o * 32 + i, pl.ds(q * 16, 16)] = vr[q]

        for lp in range(2):
            @pl.when(cid == 0)
            def _(lp=lp):
                pltpu.sync_copy((t0, t1)[lp].at[pl.ds(sid * 128, 128)], rows_all)

            @pl.when(cid == 1)
            def _(lp=lp):
                pltpu.sync_copy((t2, t3)[lp].at[pl.ds(sid * 128, 128)], rows_all)

            pltpu.sync_copy(zrows, S_sh.at[pl.ds(sid * s_stripe, s_stripe)])
            plsc.subcore_barrier()

            fill(buf_a, 0, 0)
            pltpu.async_copy(buf_a, S_sh.at[idxn_v.at[0]], sa, add=True)
            fill(buf_b, 0, 1)
            pltpu.async_copy(buf_b, S_sh.at[idxn_v.at[1]], sb, add=True)

            def body(jj, carry):
                b0 = 2 * jj
                pltpu.make_async_copy(buf_a, S_sh.at[idxn_v.at[b0 - 2]], sa).wait()
                fill(buf_a, jj, 0)
                pltpu.async_copy(buf_a, S_sh.at[idxn_v.at[b0]], sa, add=True)
                pltpu.make_async_copy(buf_b, S_sh.at[idxn_v.at[b0 - 1]], sb).wait()
                fill(buf_b, jj, 1)
                pltpu.async_copy(buf_b, S_sh.at[idxn_v.at[b0 + 1]], sb, add=True)
                return carry

            lax.fori_loop(1, blocks_per_tile // 2, body, 0)
            pltpu.make_async_copy(
                buf_a, S_sh.at[idxn_v.at[blocks_per_tile - 2]], sa).wait()
            pltpu.make_async_copy(
                buf_b, S_sh.at[idxn_v.at[blocks_per_tile - 1]], sb).wait()
            plsc.subcore_barrier()

            @pl.when(cid == 0)
            def _(lp=lp):
                pltpu.sync_copy(S_sh.at[pl.ds(sid * s_stripe, s_stripe)],
                                S_out.at[lp, pl.ds(sid * s_stripe, s_stripe)])

            @pl.when(cid == 1)
            def _(lp=lp):
                pltpu.sync_copy(S_sh.at[pl.ds(sid * s_stripe, s_stripe)],
                                S_out.at[2 + lp, pl.ds(sid * s_stripe, s_stripe)])

    return sc_scatter


def kernel(node_embeddings, hyperedge_embeddings, hyperedge_to_nodes, W_fc, b_fc, W_we, b_we, c_e):
    N, d_in = node_embeddings.shape
    H, K = hyperedge_to_nodes.shape
    d_out = W_fc.shape[0]
    assert d_in == 512 and d_out == 512

    n_tiles = 16
    stripe = 640
    NPAD = n_tiles * stripe
    PAIRS = H * K
    pairs_per_tile = PAIRS // n_tiles
    SPAD = 10112
    bw = 64
    blocks_per_tile = pairs_per_tile // bw

    f32 = jnp.float32

    eb = 256
    ht0, ht1, ht2, ht3, cc = pl.pallas_call(
        _edge_kernel,
        grid=(H // eb,),
        in_specs=[
            pl.BlockSpec((eb, d_in), lambda i: (i, 0)),
            pl.BlockSpec((d_in, d_out), lambda i: (0, 0)),
            pl.BlockSpec((1, d_out), lambda i: (0, 0)),
            pl.BlockSpec((eb, K), lambda i: (i, 0)),
        ],
        out_specs=[pl.BlockSpec((eb, 128), lambda i: (i, 0))] * 4
        + [pl.BlockSpec((eb, K), lambda i: (i, 0))],
        out_shape=[jax.ShapeDtypeStruct((H, 128), f32)] * 4
        + [jax.ShapeDtypeStruct((H, K), f32)],
    )(hyperedge_embeddings, W_fc.T, b_fc.reshape(1, -1), hyperedge_to_nodes)

    nb = 400
    s_col = pl.pallas_call(
        _node_kernel,
        grid=(N // nb,),
        in_specs=[
            pl.BlockSpec((nb, d_in), lambda i: (i, 0)),
            pl.BlockSpec((d_in, d_out), lambda i: (0, 0)),
            pl.BlockSpec((1, d_out), lambda i: (0, 0)),
            pl.BlockSpec((d_out, 1), lambda i: (0, 0)),
        ],
        out_specs=pl.BlockSpec((nb, 1), lambda i: (i, 0)),
        out_shape=jax.ShapeDtypeStruct((N, 1), f32),
    )(node_embeddings, W_we.T, b_we.reshape(1, -1), c_e.reshape(-1, 1))

    nodes3 = hyperedge_to_nodes.reshape(n_tiles, blocks_per_tile, bw)
    cntv3 = cc.reshape(n_tiles, pairs_per_tile // 128, 128)
    zrows = jnp.zeros((SPAD // n_tiles, 128), f32)
    z1 = jnp.zeros((stripe,), f32)

    sc = _make_sc_scatter(SPAD, NPAD, n_tiles, blocks_per_tile, bw, stripe,
                          pairs_per_tile)
    S_chunks, cnt = sc(nodes3, cntv3, zrows, z1, ht0, ht1, ht2, ht3)

    cb = 1000
    out = pl.pallas_call(
        functools.partial(_combine_kernel, H),
        grid=(N // cb,),
        in_specs=[
            pl.BlockSpec((4, cb, 128), lambda r: (0, r, 0)),
            pl.BlockSpec((cb, 1), lambda r: (r, 0)),
            pl.BlockSpec((cb, 1), lambda r: (r, 0)),
        ],
        out_specs=pl.BlockSpec((cb, d_out), lambda r: (r, 0)),
        out_shape=jax.ShapeDtypeStruct((N, d_out), f32),
    )(S_chunks, s_col, cnt[:N].reshape(N, 1))

    return out


# speedup vs baseline: 2.2694x; 1.0051x over previous
import functools

import jax
import jax.numpy as jnp
from jax import lax
from jax.experimental import pallas as pl
from jax.experimental.pallas import tpu as pltpu
from jax.experimental.pallas import tpu_sc as plsc


def _edge_kernel(he_ref, wt_ref, b_ref, nodes_ref, t0_ref, t1_ref, t2_ref, t3_ref, cc_ref):
    ht = jnp.dot(he_ref[...], wt_ref[...], preferred_element_type=jnp.float32) + b_ref[...]
    t0_ref[...] = ht[:, 0:128]
    t1_ref[...] = ht[:, 128:256]
    t2_ref[...] = ht[:, 256:384]
    t3_ref[...] = ht[:, 384:512]
    n = nodes_ref[...]
    rows, K = n.shape
    eq = n[:, :, None] == n[:, None, :]
    kk = jax.lax.broadcasted_iota(jnp.int32, (rows, K, K), 1)
    kp = jax.lax.broadcasted_iota(jnp.int32, (rows, K, K), 2)
    dup = jnp.any(jnp.logical_and(eq, kp < kk), axis=2)
    cc_ref[...] = 1.0 - dup.astype(jnp.float32)


def _node_kernel(x_ref, wt_ref, b_ref, ce_ref, s_ref):
    t = jnp.dot(x_ref[...], wt_ref[...], preferred_element_type=jnp.float32) + b_ref[...]
    lr = jnp.where(t >= 0, t, 0.01 * t)
    s_ref[...] = jnp.dot(lr, ce_ref[...], preferred_element_type=jnp.float32)


def _combine_kernel(H, S_ref, s_ref, cnt_ref, o_ref):
    s = s_ref[...]
    c = cnt_ref[...]
    m = jnp.maximum(s, 0.0)
    e = jnp.maximum(jnp.exp(s - m), 1e-35)
    denom = c * e + (float(H) - c) * jnp.exp(-m)
    a = e / denom
    S4 = S_ref[...]
    S = jnp.concatenate([S4[0], S4[1], S4[2], S4[3]], axis=1)
    o_ref[...] = jnp.maximum(a * S, 0.0)


def _make_sc_scatter(SPAD, NPAD, n_tiles, blocks_per_tile, bw, stripe, pairs_per_tile):
    s_stripe = SPAD // n_tiles
    mesh = plsc.VectorSubcoreMesh(core_axis_name="c", subcore_axis_name="s")

    @functools.partial(
        pl.kernel,
        mesh=mesh,
        out_type=[
            jax.ShapeDtypeStruct((4, SPAD, 128), jnp.float32),
            jax.ShapeDtypeStruct((NPAD,), jnp.float32),
        ],
        scratch_types=[
            pltpu.VMEM((blocks_per_tile, bw), jnp.int32),
            pltpu.VMEM((128, 128), jnp.float32),
            pltpu.VMEM((bw, 128), jnp.float32),
            pltpu.VMEM((bw, 128), jnp.float32),
            pltpu.VMEM_SHARED((SPAD, 128), jnp.float32),
            pltpu.VMEM_SHARED((NPAD,), jnp.float32),
            pltpu.SemaphoreType.DMA,
            pltpu.SemaphoreType.DMA,
        ],
    )
    def sc_scatter(nodes3, cntv3, zrows, z1, t0, t1, t2, t3,
                   S_out, cnt_out,
                   idxn_v, rows_all, buf_a, buf_b,
                   S_sh, cnt_sh,
                   sa, sb):
        cid = lax.axis_index("c")
        sid = lax.axis_index("s")

        pltpu.sync_copy(nodes3.at[sid], idxn_v)

        @pl.when(cid == 0)
        def _():
            pltpu.sync_copy(z1, cnt_sh.at[pl.ds(sid * stripe, stripe)])
            plsc.subcore_barrier()
            pltpu.sync_copy(cntv3.at[sid], buf_a.at[pl.ds(0, pairs_per_tile // 128)])

            def cbody(jj, carry):
                pltpu.async_copy(buf_a.at[jj, pl.ds(0, bw)],
                                 cnt_sh.at[idxn_v.at[2 * jj]], sa, add=True)
                pltpu.async_copy(buf_a.at[jj, pl.ds(bw, bw)],
                                 cnt_sh.at[idxn_v.at[2 * jj + 1]], sb, add=True)
                return carry

            lax.fori_loop(0, pairs_per_tile // 128, cbody, 0)

            def cdrain(jj, carry):
                pltpu.make_async_copy(buf_a.at[jj, pl.ds(0, bw)],
                                      cnt_sh.at[idxn_v.at[2 * jj]], sa).wait()
                pltpu.make_async_copy(buf_a.at[jj, pl.ds(bw, bw)],
                                      cnt_sh.at[idxn_v.at[2 * jj + 1]], sb).wait()
                return carry

            lax.fori_loop(0, pairs_per_tile // 128, cdrain, 0)
            plsc.subcore_barrier()
            pltpu.sync_copy(cnt_sh.at[pl.ds(sid * stripe, stripe)],
                            cnt_out.at[pl.ds(sid * stripe, stripe)])

        epb = bw // 32

        def fill(buf, jj, half):
            for o in range(epb):
                src_row = 2 * epb * jj + epb * half + o
                vr = [rows_all[src_row, pl.ds(q * 16, 16)] for q in range(8)]
                for i in range(32):
                    for q in range(8):
                        buf[o * 32 + i, pl.ds(q * 16, 16)] = vr[q]

        for lp in range(2):
            @pl.when(cid == 0)
            def _(lp=lp):
                pltpu.sync_copy((t0, t1)[lp].at[pl.ds(sid * 128, 128)], rows_all)

            @pl.when(cid == 1)
            def _(lp=lp):
                pltpu.sync_copy((t2, t3)[lp].at[pl.ds(sid * 128, 128)], rows_all)

            pltpu.sync_copy(zrows, S_sh.at[pl.ds(sid * s_stripe, s_stripe)])
            plsc.subcore_barrier()

            fill(buf_a, 0, 0)
            pltpu.async_copy(buf_a, S_sh.at[idxn_v.at[0]], sa, add=True)
            fill(buf_b, 0, 1)
            pltpu.async_copy(buf_b, S_sh.at[idxn_v.at[1]], sb, add=True)

            def body(jj, carry):
                b0 = 2 * jj
                pltpu.make_async_copy(buf_a, S_sh.at[idxn_v.at[b0 - 2]], sa).wait()
                fill(buf_a, jj, 0)
                pltpu.async_copy(buf_a, S_sh.at[idxn_v.at[b0]], sa, add=True)
                pltpu.make_async_copy(buf_b, S_sh.at[idxn_v.at[b0 - 1]], sb).wait()
                fill(buf_b, jj, 1)
                pltpu.async_copy(buf_b, S_sh.at[idxn_v.at[b0 + 1]], sb, add=True)
                return carry

            lax.fori_loop(1, blocks_per_tile // 2, body, 0)
            pltpu.make_async_copy(
                buf_a, S_sh.at[idxn_v.at[blocks_per_tile - 2]], sa).wait()
            pltpu.make_async_copy(
                buf_b, S_sh.at[idxn_v.at[blocks_per_tile - 1]], sb).wait()
            plsc.subcore_barrier()

            @pl.when(cid == 0)
            def _(lp=lp):
                pltpu.sync_copy(S_sh.at[pl.ds(sid * s_stripe, s_stripe)],
                                S_out.at[lp, pl.ds(sid * s_stripe, s_stripe)])

            @pl.when(cid == 1)
            def _(lp=lp):
                pltpu.sync_copy(S_sh.at[pl.ds(sid * s_stripe, s_stripe)],
                                S_out.at[2 + lp, pl.ds(sid * s_stripe, s_stripe)])

    return sc_scatter


def kernel(node_embeddings, hyperedge_embeddings, hyperedge_to_nodes, W_fc, b_fc, W_we, b_we, c_e):
    N, d_in = node_embeddings.shape
    H, K = hyperedge_to_nodes.shape
    d_out = W_fc.shape[0]
    assert d_in == 512 and d_out == 512

    n_tiles = 16
    stripe = 640
    NPAD = n_tiles * stripe
    PAIRS = H * K
    pairs_per_tile = PAIRS // n_tiles
    SPAD = 10112
    bw = 64
    blocks_per_tile = pairs_per_tile // bw

    f32 = jnp.float32

    eb = 256
    ht0, ht1, ht2, ht3, cc = pl.pallas_call(
        _edge_kernel,
        grid=(H // eb,),
        in_specs=[
            pl.BlockSpec((eb, d_in), lambda i: (i, 0)),
            pl.BlockSpec((d_in, d_out), lambda i: (0, 0)),
            pl.BlockSpec((1, d_out), lambda i: (0, 0)),
            pl.BlockSpec((eb, K), lambda i: (i, 0)),
        ],
        out_specs=[pl.BlockSpec((eb, 128), lambda i: (i, 0))] * 4
        + [pl.BlockSpec((eb, K), lambda i: (i, 0))],
        out_shape=[jax.ShapeDtypeStruct((H, 128), f32)] * 4
        + [jax.ShapeDtypeStruct((H, K), f32)],
    )(hyperedge_embeddings, W_fc.T, b_fc.reshape(1, -1), hyperedge_to_nodes)

    nb = 400
    s_col = pl.pallas_call(
        _node_kernel,
        grid=(N // nb,),
        in_specs=[
            pl.BlockSpec((nb, d_in), lambda i: (i, 0)),
            pl.BlockSpec((d_in, d_out), lambda i: (0, 0)),
            pl.BlockSpec((1, d_out), lambda i: (0, 0)),
            pl.BlockSpec((d_out, 1), lambda i: (0, 0)),
        ],
        out_specs=pl.BlockSpec((nb, 1), lambda i: (i, 0)),
        out_shape=jax.ShapeDtypeStruct((N, 1), f32),
    )(node_embeddings, W_we.T, b_we.reshape(1, -1), c_e.reshape(-1, 1))

    nodes3 = hyperedge_to_nodes.reshape(n_tiles, blocks_per_tile, bw)
    cntv3 = cc.reshape(n_tiles, pairs_per_tile // 128, 128)
    zrows = jnp.zeros((SPAD // n_tiles, 128), f32)
    z1 = jnp.zeros((stripe,), f32)

    sc = _make_sc_scatter(SPAD, NPAD, n_tiles, blocks_per_tile, bw, stripe,
                          pairs_per_tile)
    S_chunks, cnt = sc(nodes3, cntv3, zrows, z1, ht0, ht1, ht2, ht3)

    cb = 2000
    out = pl.pallas_call(
        functools.partial(_combine_kernel, H),
        grid=(N // cb,),
        in_specs=[
            pl.BlockSpec((4, cb, 128), lambda r: (0, r, 0)),
            pl.BlockSpec((cb, 1), lambda r: (r, 0)),
            pl.BlockSpec((cb, 1), lambda r: (r, 0)),
        ],
        out_specs=pl.BlockSpec((cb, d_out), lambda r: (r, 0)),
        out_shape=jax.ShapeDtypeStruct((N, d_out), f32),
    )(S_chunks, s_col, cnt[:N].reshape(N, 1))

    return out
